# baseline (speedup 1.0000x reference)
import functools

import jax
import jax.numpy as jnp
from jax import lax
from jax.experimental import pallas as pl
from jax.experimental.pallas import tpu as pltpu
from jax.experimental.pallas import tpu_sc as plsc

N = 10000
E = 320000
NFEAT = 128
HID = 64
W = 128

NC = 2
NS = 16
NW = NC * NS
EPW = E // NW
EPW_PAD = 10240
DCH = 128
DNCH = EPW_PAD // DCH
SCH = 128
SNCH = EPW_PAD // SCH
ACC_ROWS = 10240
ZROWS = ACC_ROWS // NS
SENT = N

_mesh = plsc.VectorSubcoreMesh(core_axis_name="c", subcore_axis_name="s")


@functools.partial(
    pl.kernel,
    out_type=jax.ShapeDtypeStruct((NC, ACC_ROWS, W), jnp.float32),
    mesh=_mesh,
    scratch_types=[
        pltpu.VMEM_SHARED((ACC_ROWS, W), jnp.float32),
        pltpu.VMEM((DNCH, DCH), jnp.int32),
        pltpu.VMEM((DCH, W), jnp.float32),
        pltpu.SemaphoreType.DMA,
    ],
)
def _deg_kernel(dst_hbm, ones_hbm, zeros_hbm, deg_hbm, acc_sp, dst_v, ones_v,
                sem):
    cid = lax.axis_index("c")
    sid = lax.axis_index("s")
    wid = cid * NS + sid
    pltpu.sync_copy(zeros_hbm, acc_sp.at[pl.ds(sid * ZROWS, ZROWS)])
    pltpu.sync_copy(ones_hbm, ones_v)
    pltpu.sync_copy(dst_hbm.at[wid], dst_v)
    plsc.subcore_barrier()

    @pl.loop(0, DNCH)
    def _(j):
        pltpu.sync_copy(ones_v, acc_sp.at[dst_v.at[j]], add=True)

    plsc.subcore_barrier()
    pltpu.sync_copy(acc_sp.at[pl.ds(sid * ZROWS, ZROWS)],
                    deg_hbm.at[cid, pl.ds(sid * ZROWS, ZROWS)])


@functools.partial(
    pl.kernel,
    out_type=jax.ShapeDtypeStruct((NC, ACC_ROWS, W), jnp.float32),
    mesh=_mesh,
    scratch_types=[
        pltpu.VMEM_SHARED((ACC_ROWS, W), jnp.float32),
        pltpu.VMEM((SNCH, SCH), jnp.int32),
        pltpu.VMEM((SNCH, SCH), jnp.int32),
        pltpu.VMEM((SCH, W), jnp.float32),
        pltpu.SemaphoreType.DMA,
    ],
)
def _segsum_kernel(hs2_hbm, src_hbm, dst_hbm, zeros_hbm, s_hbm,
                   acc_sp, src_v, dst_v, rows_v, sem):
    cid = lax.axis_index("c")
    sid = lax.axis_index("s")
    wid = cid * NS + sid
    pltpu.sync_copy(zeros_hbm, acc_sp.at[pl.ds(sid * ZROWS, ZROWS)])
    pltpu.sync_copy(src_hbm.at[wid], src_v)
    pltpu.sync_copy(dst_hbm.at[wid], dst_v)
    plsc.subcore_barrier()

    @pl.loop(0, SNCH)
    def _(j):
        pltpu.sync_copy(hs2_hbm.at[src_v.at[j]], rows_v)
        pltpu.sync_copy(rows_v, acc_sp.at[dst_v.at[j]], add=True)

    plsc.subcore_barrier()
    pltpu.sync_copy(acc_sp.at[pl.ds(sid * ZROWS, ZROWS)],
                    s_hbm.at[cid, pl.ds(sid * ZROWS, ZROWS)])


BLK = 1000


def _mm1_body(x_ref, w1_ref, b1_ref, wg_ref, o_ref):
    h = jnp.dot(x_ref[...], w1_ref[...], preferred_element_type=jnp.float32)
    h = jnp.maximum(h + b1_ref[...][None, :], 0.0)
    o_ref[...] = jnp.dot(h, wg_ref[...], preferred_element_type=jnp.float32)


_mm1 = pl.pallas_call(
    _mm1_body,
    grid=(N // BLK,),
    in_specs=[
        pl.BlockSpec((BLK, NFEAT), lambda i: (i, 0)),
        pl.BlockSpec((NFEAT, HID), lambda i: (0, 0)),
        pl.BlockSpec((HID,), lambda i: (0,)),
        pl.BlockSpec((HID, HID), lambda i: (0, 0)),
    ],
    out_specs=pl.BlockSpec((BLK, HID), lambda i: (i, 0)),
    out_shape=jax.ShapeDtypeStruct((N, HID), jnp.float32),
)


def _scale_body(deg_ref, hlin_ref, o_ref):
    deg = deg_ref[0, :, 0:1] + deg_ref[1, :, 0:1] + 1.0
    hs = hlin_ref[...] * lax.rsqrt(deg)
    o_ref[...] = jnp.concatenate([hs, hs], axis=1)


_scale = pl.pallas_call(
    _scale_body,
    grid=(N // BLK,),
    in_specs=[
        pl.BlockSpec((NC, BLK, W), lambda i: (0, i, 0)),
        pl.BlockSpec((BLK, HID), lambda i: (i, 0)),
    ],
    out_specs=pl.BlockSpec((BLK, W), lambda i: (i, 0)),
    out_shape=jax.ShapeDtypeStruct((N, W), jnp.float32),
)


def _final_body(deg_ref, s_ref, hs2_ref, bg_ref, w2_ref, b2_ref, o_ref):
    deg = deg_ref[0, :, 0:1] + deg_ref[1, :, 0:1] + 1.0
    dinv = lax.rsqrt(deg)
    g = ((s_ref[0, :, :HID] + s_ref[1, :, :HID] + hs2_ref[:, :HID]) * dinv
         + bg_ref[...][None, :])
    o = jnp.dot(g, w2_ref[...], preferred_element_type=jnp.float32)
    o_ref[...] = jnp.maximum(o + b2_ref[...][None, :], 0.0)


_final = pl.pallas_call(
    _final_body,
    grid=(N // BLK,),
    in_specs=[
        pl.BlockSpec((NC, BLK, W), lambda i: (0, i, 0)),
        pl.BlockSpec((NC, BLK, W), lambda i: (0, i, 0)),
        pl.BlockSpec((BLK, W), lambda i: (i, 0)),
        pl.BlockSpec((HID,), lambda i: (0,)),
        pl.BlockSpec((HID, HID), lambda i: (0, 0)),
        pl.BlockSpec((HID,), lambda i: (0,)),
    ],
    out_specs=pl.BlockSpec((BLK, HID), lambda i: (i, 0)),
    out_shape=jax.ShapeDtypeStruct((N, HID), jnp.float32),
)


def kernel(x, edge_index, W1, b1, Wg, bg, W2, b2):
    src = edge_index[0].reshape(NW, EPW)
    dst = edge_index[1].reshape(NW, EPW)
    pad = ((0, 0), (0, EPW_PAD - EPW))
    srcp = jnp.pad(src, pad, constant_values=0).reshape(NW, SNCH, SCH)
    dstp = jnp.pad(dst, pad, constant_values=SENT)
    dstp_deg = dstp.reshape(NW, DNCH, DCH)
    dstp_seg = dstp.reshape(NW, SNCH, SCH)
    ones = jnp.ones((DCH, W), jnp.float32)
    zeros = jnp.zeros((ZROWS, W), jnp.float32)

    deg = _deg_kernel(dstp_deg, ones, zeros)
    hlin = _mm1(x, W1, b1, Wg)
    hs2 = _scale(deg, hlin)
    s = _segsum_kernel(hs2, srcp, dstp_seg, zeros)
    return _final(deg, s, hs2, bg, W2, b2)


# speedup vs baseline: 1.8904x; 1.8904x over previous
import functools

import jax
import jax.numpy as jnp
from jax import lax
from jax.experimental import pallas as pl
from jax.experimental.pallas import tpu as pltpu
from jax.experimental.pallas import tpu_sc as plsc

N = 10000
E = 320000
NFEAT = 128
HID = 64
W = 128

NC = 2
NS = 16
NW = NC * NS
EPW = E // NW
EPW_PAD = 10240
DCH = 128
DNCH = EPW_PAD // DCH
SCH = 128
SNCH = EPW_PAD // SCH
ACC_ROWS = 10240
ZROWS = ACC_ROWS // NS
SENT = N

_mesh = plsc.VectorSubcoreMesh(core_axis_name="c", subcore_axis_name="s")


@functools.partial(
    pl.kernel,
    out_type=jax.ShapeDtypeStruct((NC, ACC_ROWS, W), jnp.float32),
    mesh=_mesh,
    scratch_types=[
        pltpu.VMEM_SHARED((ACC_ROWS, W), jnp.float32),
        pltpu.VMEM((DNCH, DCH), jnp.int32),
        pltpu.VMEM((DCH, W), jnp.float32),
        pltpu.SemaphoreType.DMA,
    ],
)
def _deg_kernel(dst_hbm, ones_hbm, zeros_hbm, deg_hbm, acc_sp, dst_v, ones_v,
                sem):
    cid = lax.axis_index("c")
    sid = lax.axis_index("s")
    wid = cid * NS + sid
    pltpu.sync_copy(zeros_hbm, acc_sp.at[pl.ds(sid * ZROWS, ZROWS)])
    pltpu.sync_copy(ones_hbm, ones_v)
    pltpu.sync_copy(dst_hbm.at[wid], dst_v)
    plsc.subcore_barrier()

    @pl.loop(0, DNCH)
    def _(j):
        pltpu.sync_copy(ones_v, acc_sp.at[dst_v.at[j]], add=True)

    plsc.subcore_barrier()
    pltpu.sync_copy(acc_sp.at[pl.ds(sid * ZROWS, ZROWS)],
                    deg_hbm.at[cid, pl.ds(sid * ZROWS, ZROWS)])


@functools.partial(
    pl.kernel,
    out_type=jax.ShapeDtypeStruct((NC, ACC_ROWS, W), jnp.float32),
    mesh=_mesh,
    scratch_types=[
        pltpu.VMEM_SHARED((ACC_ROWS, W), jnp.float32),
        pltpu.VMEM((SNCH, SCH), jnp.int32),
        pltpu.VMEM((SNCH, SCH), jnp.int32),
        pltpu.VMEM((SCH, W), jnp.float32),
        pltpu.SemaphoreType.DMA,
    ],
)
def _segsum_kernel(hs2_hbm, src_hbm, dst_hbm, zeros_hbm, s_hbm,
                   acc_sp, src_v, dst_v, rows_v, sem):
    cid = lax.axis_index("c")
    sid = lax.axis_index("s")
    wid = cid * NS + sid
    pltpu.sync_copy(zeros_hbm, acc_sp.at[pl.ds(sid * ZROWS, ZROWS)])
    pltpu.sync_copy(src_hbm.at[wid], src_v)
    pltpu.sync_copy(dst_hbm.at[wid], dst_v)
    plsc.subcore_barrier()

    @pl.loop(0, SNCH)
    def _(j):
        pltpu.sync_copy(hs2_hbm.at[src_v.at[j]], rows_v)
        pltpu.sync_copy(rows_v, acc_sp.at[dst_v.at[j]], add=True)

    plsc.subcore_barrier()
    pltpu.sync_copy(acc_sp.at[pl.ds(sid * ZROWS, ZROWS)],
                    s_hbm.at[cid, pl.ds(sid * ZROWS, ZROWS)])


BLK = 1000


def _mm1_body(x_ref, w1_ref, b1_ref, wg_ref, o_ref):
    h = jnp.dot(x_ref[...], w1_ref[...], preferred_element_type=jnp.float32)
    h = jnp.maximum(h + b1_ref[...][None, :], 0.0)
    o_ref[...] = jnp.dot(h, wg_ref[...], preferred_element_type=jnp.float32)


_mm1 = pl.pallas_call(
    _mm1_body,
    grid=(N // BLK,),
    in_specs=[
        pl.BlockSpec((BLK, NFEAT), lambda i: (i, 0)),
        pl.BlockSpec((NFEAT, HID), lambda i: (0, 0)),
        pl.BlockSpec((HID,), lambda i: (0,)),
        pl.BlockSpec((HID, HID), lambda i: (0, 0)),
    ],
    out_specs=pl.BlockSpec((BLK, HID), lambda i: (i, 0)),
    out_shape=jax.ShapeDtypeStruct((N, HID), jnp.float32),
)


def _scale_body(deg_ref, hlin_ref, o_ref):
    deg = deg_ref[0, :, 0:1] + deg_ref[1, :, 0:1] + 1.0
    hs = hlin_ref[...] * lax.rsqrt(deg)
    o_ref[...] = jnp.concatenate([hs, hs], axis=1)


_scale = pl.pallas_call(
    _scale_body,
    grid=(N // BLK,),
    in_specs=[
        pl.BlockSpec((NC, BLK, W), lambda i: (0, i, 0)),
        pl.BlockSpec((BLK, HID), lambda i: (i, 0)),
    ],
    out_specs=pl.BlockSpec((BLK, W), lambda i: (i, 0)),
    out_shape=jax.ShapeDtypeStruct((N, W), jnp.float32),
)


def _final_body(deg_ref, s_ref, hs2_ref, bg_ref, w2_ref, b2_ref, o_ref):
    deg = deg_ref[0, :, 0:1] + deg_ref[1, :, 0:1] + 1.0
    dinv = lax.rsqrt(deg)
    g = ((s_ref[0, :, :HID] + s_ref[1, :, :HID] + hs2_ref[:, :HID]) * dinv
         + bg_ref[...][None, :])
    o = jnp.dot(g, w2_ref[...], preferred_element_type=jnp.float32)
    o_ref[...] = jnp.maximum(o + b2_ref[...][None, :], 0.0)


_final = pl.pallas_call(
    _final_body,
    grid=(N // BLK,),
    in_specs=[
        pl.BlockSpec((NC, BLK, W), lambda i: (0, i, 0)),
        pl.BlockSpec((NC, BLK, W), lambda i: (0, i, 0)),
        pl.BlockSpec((BLK, W), lambda i: (i, 0)),
        pl.BlockSpec((HID,), lambda i: (0,)),
        pl.BlockSpec((HID, HID), lambda i: (0, 0)),
        pl.BlockSpec((HID,), lambda i: (0,)),
    ],
    out_specs=pl.BlockSpec((BLK, HID), lambda i: (i, 0)),
    out_shape=jax.ShapeDtypeStruct((N, HID), jnp.float32),
)


def kernel(x, edge_index, W1, b1, Wg, bg, W2, b2):
    src = edge_index[0].reshape(NW, EPW)
    dst = edge_index[1].reshape(NW, EPW)
    npad = EPW_PAD - EPW
    pad_src = jnp.broadcast_to((jnp.arange(npad, dtype=jnp.int32) * 37) % N,
                               (NW, npad))
    pad_dst = jnp.broadcast_to(SENT + (jnp.arange(npad, dtype=jnp.int32)
                                       % (ACC_ROWS - N)), (NW, npad))
    srcp = jnp.concatenate([src, pad_src], axis=1).reshape(NW, SNCH, SCH)
    dstp = jnp.concatenate([dst, pad_dst], axis=1)
    dstp_deg = dstp.reshape(NW, DNCH, DCH)
    dstp_seg = dstp.reshape(NW, SNCH, SCH)
    ones = jnp.ones((DCH, W), jnp.float32)
    zeros = jnp.zeros((ZROWS, W), jnp.float32)

    deg = _deg_kernel(dstp_deg, ones, zeros)
    hlin = _mm1(x, W1, b1, Wg)
    hs2 = _scale(deg, hlin)
    s = _segsum_kernel(hs2, srcp, dstp_seg, zeros)
    return _final(deg, s, hs2, bg, W2, b2)


# speedup vs baseline: 2.1907x; 1.1589x over previous
import functools

import jax
import jax.numpy as jnp
from jax import lax
from jax.experimental import pallas as pl
from jax.experimental.pallas import tpu as pltpu
from jax.experimental.pallas import tpu_sc as plsc

N = 10000
E = 320000
NFEAT = 128
HID = 64
W = 128

NC = 2
NS = 16
NW = NC * NS
EPW = E // NW
EPW_PAD = 10240
DCH = 128
DNCH = EPW_PAD // DCH
SCH = 64
SNCH = EPW_PAD // SCH
ACC_ROWS = 10240
ZROWS = ACC_ROWS // NS
SENT = N

_mesh = plsc.VectorSubcoreMesh(core_axis_name="c", subcore_axis_name="s")


@functools.partial(
    pl.kernel,
    out_type=jax.ShapeDtypeStruct((NC, ACC_ROWS, W), jnp.float32),
    mesh=_mesh,
    scratch_types=[
        pltpu.VMEM_SHARED((ACC_ROWS, W), jnp.float32),
        pltpu.VMEM((DNCH, DCH), jnp.int32),
        pltpu.VMEM((DCH, W), jnp.float32),
        pltpu.SemaphoreType.DMA,
    ],
)
def _deg_kernel(dst_hbm, ones_hbm, zeros_hbm, deg_hbm, acc_sp, dst_v, ones_v,
                sem):
    cid = lax.axis_index("c")
    sid = lax.axis_index("s")
    wid = cid * NS + sid
    pltpu.sync_copy(zeros_hbm, acc_sp.at[pl.ds(sid * ZROWS, ZROWS)])
    pltpu.sync_copy(ones_hbm, ones_v)
    pltpu.sync_copy(dst_hbm.at[wid], dst_v)
    plsc.subcore_barrier()

    @pl.loop(0, DNCH)
    def _(j):
        pltpu.sync_copy(ones_v, acc_sp.at[dst_v.at[j]], add=True)

    plsc.subcore_barrier()
    pltpu.sync_copy(acc_sp.at[pl.ds(sid * ZROWS, ZROWS)],
                    deg_hbm.at[cid, pl.ds(sid * ZROWS, ZROWS)])


@functools.partial(
    pl.kernel,
    out_type=jax.ShapeDtypeStruct((NC, ACC_ROWS, W), jnp.float32),
    mesh=_mesh,
    scratch_types=[
        pltpu.VMEM_SHARED((ACC_ROWS, W), jnp.float32),
        pltpu.VMEM((SNCH // 2, SCH), jnp.int32),
        pltpu.VMEM((SNCH // 2, SCH), jnp.int32),
        pltpu.VMEM((SCH, W), jnp.float32),
        pltpu.VMEM((SCH, W), jnp.float32),
        pltpu.SemaphoreType.DMA,
        pltpu.SemaphoreType.DMA,
    ],
)
def _segsum_kernel(hs2_hbm, src_hbm, dst_hbm, zeros_hbm, s_hbm,
                   acc_sp, src_v, dst_v, rows0, rows1, sem0, sem1):
    cid = lax.axis_index("c")
    sid = lax.axis_index("s")
    wid = cid * NS + sid
    pltpu.sync_copy(zeros_hbm, acc_sp.at[pl.ds(sid * ZROWS, ZROWS)])
    plsc.subcore_barrier()

    PH = SNCH // 2
    for phase in range(2):
        pltpu.sync_copy(src_hbm.at[wid, pl.ds(phase * PH, PH)], src_v)
        pltpu.sync_copy(dst_hbm.at[wid, pl.ds(phase * PH, PH)], dst_v)
        pltpu.async_copy(hs2_hbm.at[src_v.at[0]], rows0, sem0)

        @pl.loop(0, PH, step=2)
        def _(j):
            pltpu.async_copy(hs2_hbm.at[src_v.at[j + 1]], rows1, sem1)
            pltpu.make_async_copy(hs2_hbm.at[src_v.at[j]], rows0, sem0).wait()
            pltpu.sync_copy(rows0, acc_sp.at[dst_v.at[j]], add=True)

            @pl.when(j + 2 < PH)
            def _():
                pltpu.async_copy(hs2_hbm.at[src_v.at[j + 2]], rows0, sem0)

            pltpu.make_async_copy(hs2_hbm.at[src_v.at[j + 1]], rows1, sem1).wait()
            pltpu.sync_copy(rows1, acc_sp.at[dst_v.at[j + 1]], add=True)

    plsc.subcore_barrier()
    pltpu.sync_copy(acc_sp.at[pl.ds(sid * ZROWS, ZROWS)],
                    s_hbm.at[cid, pl.ds(sid * ZROWS, ZROWS)])


BLK = 1000


def _mm1_body(x_ref, w1_ref, b1_ref, wg_ref, o_ref):
    h = jnp.dot(x_ref[...], w1_ref[...], preferred_element_type=jnp.float32)
    h = jnp.maximum(h + b1_ref[...][None, :], 0.0)
    o_ref[...] = jnp.dot(h, wg_ref[...], preferred_element_type=jnp.float32)


_mm1 = pl.pallas_call(
    _mm1_body,
    grid=(N // BLK,),
    in_specs=[
        pl.BlockSpec((BLK, NFEAT), lambda i: (i, 0)),
        pl.BlockSpec((NFEAT, HID), lambda i: (0, 0)),
        pl.BlockSpec((HID,), lambda i: (0,)),
        pl.BlockSpec((HID, HID), lambda i: (0, 0)),
    ],
    out_specs=pl.BlockSpec((BLK, HID), lambda i: (i, 0)),
    out_shape=jax.ShapeDtypeStruct((N, HID), jnp.float32),
)


def _scale_body(deg_ref, hlin_ref, o_ref):
    deg = deg_ref[0, :, 0:1] + deg_ref[1, :, 0:1] + 1.0
    hs = hlin_ref[...] * lax.rsqrt(deg)
    o_ref[...] = jnp.concatenate([hs, hs], axis=1)


_scale = pl.pallas_call(
    _scale_body,
    grid=(N // BLK,),
    in_specs=[
        pl.BlockSpec((NC, BLK, W), lambda i: (0, i, 0)),
        pl.BlockSpec((BLK, HID), lambda i: (i, 0)),
    ],
    out_specs=pl.BlockSpec((BLK, W), lambda i: (i, 0)),
    out_shape=jax.ShapeDtypeStruct((N, W), jnp.float32),
)


def _final_body(deg_ref, s_ref, hs2_ref, bg_ref, w2_ref, b2_ref, o_ref):
    deg = deg_ref[0, :, 0:1] + deg_ref[1, :, 0:1] + 1.0
    dinv = lax.rsqrt(deg)
    g = ((s_ref[0, :, :HID] + s_ref[1, :, :HID] + hs2_ref[:, :HID]) * dinv
         + bg_ref[...][None, :])
    o = jnp.dot(g, w2_ref[...], preferred_element_type=jnp.float32)
    o_ref[...] = jnp.maximum(o + b2_ref[...][None, :], 0.0)


_final = pl.pallas_call(
    _final_body,
    grid=(N // BLK,),
    in_specs=[
        pl.BlockSpec((NC, BLK, W), lambda i: (0, i, 0)),
        pl.BlockSpec((NC, BLK, W), lambda i: (0, i, 0)),
        pl.BlockSpec((BLK, W), lambda i: (i, 0)),
        pl.BlockSpec((HID,), lambda i: (0,)),
        pl.BlockSpec((HID, HID), lambda i: (0, 0)),
        pl.BlockSpec((HID,), lambda i: (0,)),
    ],
    out_specs=pl.BlockSpec((BLK, HID), lambda i: (i, 0)),
    out_shape=jax.ShapeDtypeStruct((N, HID), jnp.float32),
)


def kernel(x, edge_index, W1, b1, Wg, bg, W2, b2):
    src = edge_index[0].reshape(NW, EPW)
    dst = edge_index[1].reshape(NW, EPW)
    npad = EPW_PAD - EPW
    pad_src = jnp.broadcast_to((jnp.arange(npad, dtype=jnp.int32) * 37) % N,
                               (NW, npad))
    pad_dst = jnp.broadcast_to(SENT + (jnp.arange(npad, dtype=jnp.int32)
                                       % (ACC_ROWS - N)), (NW, npad))
    srcp = jnp.concatenate([src, pad_src], axis=1).reshape(NW, SNCH, SCH)
    dstp = jnp.concatenate([dst, pad_dst], axis=1)
    dstp_deg = dstp.reshape(NW, DNCH, DCH)
    dstp_seg = dstp.reshape(NW, SNCH, SCH)
    ones = jnp.ones((DCH, W), jnp.float32)
    zeros = jnp.zeros((ZROWS, W), jnp.float32)

    deg = _deg_kernel(dstp_deg, ones, zeros)
    hlin = _mm1(x, W1, b1, Wg)
    hs2 = _scale(deg, hlin)
    s = _segsum_kernel(hs2, srcp, dstp_seg, zeros)
    return _final(deg, s, hs2, bg, W2, b2)
